# baseline (speedup 1.0000x reference)
import functools

import jax
import jax.numpy as jnp
from jax import lax
from jax.experimental import pallas as pl
from jax.experimental.pallas import tpu as pltpu
from jax.experimental.pallas import tpu_sc as plsc

V = 100000
D = 128
L = 200


def _sc_body(ctx_hbm, tab_hbm, out_hbm, idx_v, rows_v, vsum_v, sem):
    cid = lax.axis_index("c")
    sid = lax.axis_index("s")

    @pl.when(jnp.logical_and(cid == 0, sid == 0))
    def _():
        pltpu.sync_copy(ctx_hbm, idx_v)
        cp0 = pltpu.async_copy(
            tab_hbm.at[idx_v.at[pl.ds(0, 128)]], rows_v.at[pl.ds(0, 128)], sem
        )
        cp1 = pltpu.async_copy(
            tab_hbm.at[idx_v.at[pl.ds(128, L - 128)]],
            rows_v.at[pl.ds(128, L - 128)],
            sem,
        )
        cp0.wait()
        cp1.wait()

        def sum_body(i, acc):
            return tuple(
                acc[j] + rows_v[i, pl.ds(j * 16, 16)] for j in range(D // 16)
            )

        acc = lax.fori_loop(
            0,
            L,
            sum_body,
            tuple(jnp.zeros((16,), jnp.float32) for _ in range(D // 16)),
        )
        scale = jnp.float32(1.0 / L)
        for j in range(D // 16):
            vsum_v[pl.ds(j * 16, 16)] = acc[j] * scale
        pltpu.sync_copy(vsum_v, out_hbm)


@functools.cache
def _sc_gather_mean():
    return pl.kernel(
        _sc_body,
        out_type=jax.ShapeDtypeStruct((D,), jnp.float32),
        mesh=plsc.VectorSubcoreMesh(core_axis_name="c", subcore_axis_name="s"),
        scratch_types=[
            pltpu.VMEM((L,), jnp.int32),
            pltpu.VMEM((L, D), jnp.float32),
            pltpu.VMEM((D,), jnp.float32),
            pltpu.SemaphoreType.DMA,
        ],
    )

BLK = 8192
NB = -(-V // BLK)


def _tc1_body(v_ref, w_ref, b_ref, logits_ref, lse_ref, acc_ref):
    i = pl.program_id(0)

    @pl.when(i == 0)
    def _():
        acc_ref[0] = -jnp.inf
        acc_ref[1] = 0.0

    logits = (
        lax.dot_general(
            v_ref[...],
            w_ref[...],
            (((1,), (1,)), ((), ())),
            preferred_element_type=jnp.float32,
        )
        + b_ref[...]
    )
    logits_ref[...] = logits

    pos = lax.broadcasted_iota(jnp.int32, (1, BLK), 1) + i * BLK
    valid = pos < V
    bmax = jnp.max(jnp.where(valid, logits, -jnp.inf))
    m_old = acc_ref[0]
    s_old = acc_ref[1]
    m_new = jnp.maximum(m_old, bmax)
    s_new = s_old * jnp.exp(m_old - m_new) + jnp.sum(
        jnp.where(valid, jnp.exp(logits - m_new), 0.0)
    )
    acc_ref[0] = m_new
    acc_ref[1] = s_new

    @pl.when(i == NB - 1)
    def _():
        lse_ref[0, 0] = m_new + jnp.log(s_new)


_tc_matvec_lse = pl.pallas_call(
    _tc1_body,
    grid=(NB,),
    in_specs=[
        pl.BlockSpec((1, D), lambda i: (0, 0)),
        pl.BlockSpec((BLK, D), lambda i: (i, 0)),
        pl.BlockSpec((1, BLK), lambda i: (0, i)),
    ],
    out_specs=[
        pl.BlockSpec((1, BLK), lambda i: (0, i)),
        pl.BlockSpec(memory_space=pltpu.SMEM),
    ],
    out_shape=[
        jax.ShapeDtypeStruct((1, V), jnp.float32),
        jax.ShapeDtypeStruct((1, 1), jnp.float32),
    ],
    scratch_shapes=[pltpu.SMEM((2,), jnp.float32)],
    compiler_params=pltpu.CompilerParams(
        dimension_semantics=("arbitrary",)
    ),
)


def _tc2_body(logits_ref, lse_ref, out_ref):
    out_ref[...] = logits_ref[...] - lse_ref[0, 0]


_tc_normalize = pl.pallas_call(
    _tc2_body,
    grid=(NB,),
    in_specs=[
        pl.BlockSpec((1, BLK), lambda i: (0, i)),
        pl.BlockSpec(memory_space=pltpu.SMEM),
    ],
    out_specs=pl.BlockSpec((1, BLK), lambda i: (0, i)),
    out_shape=jax.ShapeDtypeStruct((1, V), jnp.float32),
)


def kernel(context, emb_table, W, b):
    context = context.astype(jnp.int32)
    v = _sc_gather_mean()(context, emb_table)
    logits, lse = _tc_matvec_lse(v.reshape(1, D), W, b.reshape(1, V))
    return _tc_normalize(logits, lse)


# speedup vs baseline: 1.2021x; 1.2021x over previous
import functools

import jax
import jax.numpy as jnp
from jax import lax
from jax.experimental import pallas as pl
from jax.experimental.pallas import tpu as pltpu
from jax.experimental.pallas import tpu_sc as plsc

V = 100000
D = 128
L = 200


def _sc_body(ctx_hbm, tab_hbm, out_hbm, idx_v, rows_v, vsum_v, sem):
    cid = lax.axis_index("c")
    sid = lax.axis_index("s")

    @pl.when(jnp.logical_and(cid == 0, sid == 0))
    def _():
        pltpu.sync_copy(ctx_hbm, idx_v)
        cp0 = pltpu.async_copy(
            tab_hbm.at[idx_v.at[pl.ds(0, 128)]], rows_v.at[pl.ds(0, 128)], sem
        )
        cp1 = pltpu.async_copy(
            tab_hbm.at[idx_v.at[pl.ds(128, L - 128)]],
            rows_v.at[pl.ds(128, L - 128)],
            sem,
        )
        cp0.wait()
        cp1.wait()

        def sum_body(i, acc):
            return tuple(
                acc[j] + rows_v[i, pl.ds(j * 16, 16)] for j in range(D // 16)
            )

        acc = lax.fori_loop(
            0,
            L,
            sum_body,
            tuple(jnp.zeros((16,), jnp.float32) for _ in range(D // 16)),
        )
        scale = jnp.float32(1.0 / L)
        for j in range(D // 16):
            vsum_v[pl.ds(j * 16, 16)] = acc[j] * scale
        pltpu.sync_copy(vsum_v, out_hbm)


@functools.cache
def _sc_gather_mean():
    return pl.kernel(
        _sc_body,
        out_type=jax.ShapeDtypeStruct((D,), jnp.float32),
        mesh=plsc.VectorSubcoreMesh(core_axis_name="c", subcore_axis_name="s"),
        scratch_types=[
            pltpu.VMEM((L,), jnp.int32),
            pltpu.VMEM((L, D), jnp.float32),
            pltpu.VMEM((D,), jnp.float32),
            pltpu.SemaphoreType.DMA,
        ],
    )

BLK = 6400
NB = -(-V // BLK)
NH = NB // 2


def _block_stats(logits, base):
    pos = lax.broadcasted_iota(jnp.int32, (1, BLK), 1) + base
    valid = pos < V
    bmax = jnp.max(jnp.where(valid, logits, -jnp.inf))
    return valid, bmax


def _tc1_body(v_ref, w1_ref, w2_ref, b1_ref, b2_ref, out_ref, acc_ref):
    i = pl.program_id(0)

    @pl.when(i == 0)
    def _():
        acc_ref[0] = -jnp.inf
        acc_ref[1] = 0.0

    v = v_ref[...]
    dn = (((1,), (1,)), ((), ()))
    lo1 = lax.dot_general(v, w1_ref[...], dn, preferred_element_type=jnp.float32)
    lo1 = lo1 + b1_ref[...]
    lo2 = lax.dot_general(v, w2_ref[...], dn, preferred_element_type=jnp.float32)
    lo2 = lo2 + b2_ref[...]
    out_ref[:, pl.ds(i * BLK, BLK)] = lo1
    out_ref[:, pl.ds((i + NH) * BLK, BLK)] = lo2

    valid1, bmax1 = _block_stats(lo1, i * BLK)
    valid2, bmax2 = _block_stats(lo2, (i + NH) * BLK)
    m_old = acc_ref[0]
    s_old = acc_ref[1]
    m_new = jnp.maximum(m_old, jnp.maximum(bmax1, bmax2))
    s_new = (
        s_old * jnp.exp(m_old - m_new)
        + jnp.sum(jnp.where(valid1, jnp.exp(lo1 - m_new), 0.0))
        + jnp.sum(jnp.where(valid2, jnp.exp(lo2 - m_new), 0.0))
    )
    acc_ref[0] = m_new
    acc_ref[1] = s_new

    @pl.when(i == NH - 1)
    def _():
        lse = m_new + jnp.log(s_new)
        out_ref[...] = out_ref[...] - lse


_tc_matvec_lse = pl.pallas_call(
    _tc1_body,
    grid=(NH,),
    in_specs=[
        pl.BlockSpec((1, D), lambda i: (0, 0)),
        pl.BlockSpec((BLK, D), lambda i: (i, 0)),
        pl.BlockSpec((BLK, D), lambda i: (i + NH, 0)),
        pl.BlockSpec((1, BLK), lambda i: (0, i)),
        pl.BlockSpec((1, BLK), lambda i: (0, i + NH)),
    ],
    out_specs=pl.BlockSpec((1, NB * BLK), lambda i: (0, 0)),
    out_shape=jax.ShapeDtypeStruct((1, NB * BLK), jnp.float32),
    scratch_shapes=[
        pltpu.SMEM((2,), jnp.float32),
    ],
    compiler_params=pltpu.CompilerParams(
        dimension_semantics=("arbitrary",)
    ),
)


def kernel(context, emb_table, W, b):
    context = context.astype(jnp.int32)
    v = _sc_gather_mean()(context, emb_table)
    b2 = b.reshape(1, V)
    padded = _tc_matvec_lse(v.reshape(1, D), W, W, b2, b2)
    return padded[:, :V]


# speedup vs baseline: 1.2098x; 1.0064x over previous
import functools

import jax
import jax.numpy as jnp
from jax import lax
from jax.experimental import pallas as pl
from jax.experimental.pallas import tpu as pltpu
from jax.experimental.pallas import tpu_sc as plsc

V = 100000
D = 128
L = 200


def _sc_body(ctx_hbm, tab_hbm, out_hbm, idx_v, rows_v, vsum_v, sem):
    cid = lax.axis_index("c")
    sid = lax.axis_index("s")

    @pl.when(jnp.logical_and(cid == 0, sid == 0))
    def _():
        pltpu.sync_copy(ctx_hbm, idx_v)
        cp0 = pltpu.async_copy(
            tab_hbm.at[idx_v.at[pl.ds(0, 128)]], rows_v.at[pl.ds(0, 128)], sem
        )
        cp1 = pltpu.async_copy(
            tab_hbm.at[idx_v.at[pl.ds(128, L - 128)]],
            rows_v.at[pl.ds(128, L - 128)],
            sem,
        )
        cp0.wait()
        cp1.wait()

        def sum_body(i, acc):
            return tuple(
                acc[j] + rows_v[i, pl.ds(j * 16, 16)] for j in range(D // 16)
            )

        acc = lax.fori_loop(
            0,
            L,
            sum_body,
            tuple(jnp.zeros((16,), jnp.float32) for _ in range(D // 16)),
        )
        scale = jnp.float32(1.0 / L)
        for j in range(D // 16):
            vsum_v[pl.ds(j * 16, 16)] = acc[j] * scale
        pltpu.sync_copy(vsum_v, out_hbm)


@functools.cache
def _sc_gather_mean():
    return pl.kernel(
        _sc_body,
        out_type=jax.ShapeDtypeStruct((D,), jnp.float32),
        mesh=plsc.VectorSubcoreMesh(core_axis_name="c", subcore_axis_name="s"),
        scratch_types=[
            pltpu.VMEM((L,), jnp.int32),
            pltpu.VMEM((L, D), jnp.float32),
            pltpu.VMEM((D,), jnp.float32),
            pltpu.SemaphoreType.DMA,
        ],
    )

BLK = 3200
NS = 4
NB = -(-V // BLK)
NH = NB // NS
assert NB % NS == 0


def _tc1_body(*refs):
    v_ref = refs[0]
    w_refs = refs[1 : 1 + NS]
    b_refs = refs[1 + NS : 1 + 2 * NS]
    out_ref = refs[1 + 2 * NS]
    acc_ref = refs[2 + 2 * NS]
    i = pl.program_id(0)

    @pl.when(i == 0)
    def _():
        acc_ref[0] = -jnp.inf
        acc_ref[1] = 0.0

    v = v_ref[...]
    dn = (((1,), (1,)), ((), ()))
    los, valids, bmaxs = [], [], []
    for k in range(NS):
        lo = lax.dot_general(
            v, w_refs[k][...], dn, preferred_element_type=jnp.float32
        )
        lo = lo + b_refs[k][...]
        out_ref[:, pl.ds((i + k * NH) * BLK, BLK)] = lo
        pos = lax.broadcasted_iota(jnp.int32, (1, BLK), 1) + (i + k * NH) * BLK
        valid = pos < V
        los.append(lo)
        valids.append(valid)
        bmaxs.append(jnp.max(jnp.where(valid, lo, -jnp.inf)))

    m_old = acc_ref[0]
    s_old = acc_ref[1]
    m_new = m_old
    for bm in bmaxs:
        m_new = jnp.maximum(m_new, bm)
    s_new = s_old * jnp.exp(m_old - m_new)
    for lo, valid in zip(los, valids):
        s_new = s_new + jnp.sum(jnp.where(valid, jnp.exp(lo - m_new), 0.0))
    acc_ref[0] = m_new
    acc_ref[1] = s_new

    @pl.when(i == NH - 1)
    def _():
        lse = m_new + jnp.log(s_new)
        out_ref[...] = out_ref[...] - lse


def _mk_w_spec(k):
    return pl.BlockSpec((BLK, D), lambda i, k=k: (i + k * NH, 0))


def _mk_b_spec(k):
    return pl.BlockSpec((1, BLK), lambda i, k=k: (0, i + k * NH))


_tc_matvec_lse = pl.pallas_call(
    _tc1_body,
    grid=(NH,),
    in_specs=(
        [pl.BlockSpec((1, D), lambda i: (0, 0))]
        + [_mk_w_spec(k) for k in range(NS)]
        + [_mk_b_spec(k) for k in range(NS)]
    ),
    out_specs=pl.BlockSpec((1, NB * BLK), lambda i: (0, 0)),
    out_shape=jax.ShapeDtypeStruct((1, NB * BLK), jnp.float32),
    scratch_shapes=[
        pltpu.SMEM((2,), jnp.float32),
    ],
    compiler_params=pltpu.CompilerParams(
        dimension_semantics=("arbitrary",)
    ),
)


def kernel(context, emb_table, W, b):
    context = context.astype(jnp.int32)
    v = _sc_gather_mean()(context, emb_table)
    b2 = b.reshape(1, V)
    padded = _tc_matvec_lse(
        v.reshape(1, D), *([W] * NS), *([b2] * NS)
    )
    return padded[:, :V]
